# baseline (speedup 1.0000x reference)
import functools
import jax
import jax.numpy as jnp
from jax import lax
from jax.experimental import pallas as pl
from jax.experimental.pallas import tpu as pltpu
from jax.experimental.pallas import tpu_sc as plsc

N_U = 10000
N_V = 10000
E = 320000
D_IN = 128
D_OUT = 128
NS = 4
D_SUB = D_OUT // NS
E_BLK = E // NS
C = 128
CHUNKS = E_BLK // C
TOT = NS * CHUNKS
N_TILES = 16
CBIG = 157
CSML = 156
NR0 = 624
NRL = 640


def _tc_matmul(xu_ref, xv_ref, w_ref, out_ref):
    d = pl.program_id(0)

    @pl.when(d == 0)
    def _():
        out_ref[0] = jnp.dot(xv_ref[...], w_ref[...],
                             preferred_element_type=jnp.float32)

    @pl.when(d == 1)
    def _():
        out_ref[0] = jnp.dot(xu_ref[...], w_ref[...],
                             preferred_element_type=jnp.float32)


_SC_MESH = plsc.VectorSubcoreMesh(core_axis_name="c", subcore_axis_name="s")


@functools.partial(
    pl.kernel,
    out_type=(jax.ShapeDtypeStruct((N_U, D_OUT), jnp.float32),
              jax.ShapeDtypeStruct((N_V, D_OUT), jnp.float32)),
    mesh=_SC_MESH,
    scratch_types=[
        pltpu.VMEM_SHARED((NS, N_U, D_SUB), jnp.float32),
        pltpu.VMEM((8, C), jnp.int32),
        pltpu.VMEM((8, C), jnp.int32),
        pltpu.VMEM((8, C), jnp.float32),
        pltpu.VMEM((4, C, D_SUB), jnp.float32),
        pltpu.VMEM((2, C, D_SUB), jnp.float32),
        pltpu.VMEM((NRL, D_SUB), jnp.float32),
        pltpu.SemaphoreType.DMA((4,)),
        pltpu.SemaphoreType.DMA((2,)),
        pltpu.SemaphoreType.DMA((8, 3)),
    ],
    compiler_params=pltpu.CompilerParams(use_tc_tiling_on_sc=False),
)
def _sc_aggregate(t_hbm, eu_hbm, ev_hbm, geu_hbm, gev_hbm, s0_hbm, s1_hbm,
                  outu_hbm, outv_hbm,
                  acc, gidx_a, sidx_a, vals_a, rows, scbuf, buf,
                  sems, ssems, stsems):
    d = lax.axis_index("c")
    t = lax.axis_index("s")

    zvec = jnp.zeros((16,), jnp.float32)
    start = t * NR0
    nrows = jnp.where(t == N_TILES - 1, NRL, NR0)
    cstart = t * CSML + jnp.minimum(t, 4)
    nct = jnp.where(t < 4, CBIG, CSML)

    def _pref(g):
        slot = lax.rem(g, 8)
        row = cstart + g

        @pl.when(d == 0)
        def _():
            pltpu.async_copy(gev_hbm.at[row], gidx_a.at[slot],
                             stsems.at[slot, 0])
            pltpu.async_copy(eu_hbm.at[row], sidx_a.at[slot],
                             stsems.at[slot, 1])
            pltpu.async_copy(s0_hbm.at[row], vals_a.at[slot],
                             stsems.at[slot, 2])

        @pl.when(d == 1)
        def _():
            pltpu.async_copy(geu_hbm.at[row], gidx_a.at[slot],
                             stsems.at[slot, 0])
            pltpu.async_copy(ev_hbm.at[row], sidx_a.at[slot],
                             stsems.at[slot, 1])
            pltpu.async_copy(s1_hbm.at[row], vals_a.at[slot],
                             stsems.at[slot, 2])

    def _pref_wait(g):
        slot = lax.rem(g, 8)
        pltpu.make_async_copy(eu_hbm.at[0], gidx_a.at[slot],
                              stsems.at[slot, 0]).wait()
        pltpu.make_async_copy(eu_hbm.at[0], sidx_a.at[slot],
                              stsems.at[slot, 1]).wait()
        pltpu.make_async_copy(s0_hbm.at[0], vals_a.at[slot],
                              stsems.at[slot, 2]).wait()

    for j in range(4):
        _pref(j)

    def _zero(j, carry):
        buf[j, pl.ds(0, 16)] = zvec
        buf[j, pl.ds(16, 16)] = zvec
        return carry
    lax.fori_loop(0, NRL, _zero, 0)
    for i in range(NS):
        @pl.when(t < N_TILES - 1)
        def _():
            pltpu.sync_copy(buf.at[pl.ds(0, NR0)],
                            acc.at[i].at[pl.ds(start, NR0)])

        @pl.when(t == N_TILES - 1)
        def _():
            pltpu.sync_copy(buf, acc.at[i].at[pl.ds(start, NRL)])
    plsc.subcore_barrier()

    table = t_hbm.at[d]
    for j in range(3):
        _pref_wait(j)
        pltpu.async_copy(table.at[gidx_a.at[j]], rows.at[j], sems.at[j])

    def _chunk(g, carry):
        b = lax.rem(g, 4)
        sb = lax.rem(g, 2)
        slot = lax.rem(g, 8)
        i = lax.div(cstart + g, CHUNKS)

        @pl.when(g + 4 < nct)
        def _():
            _pref(g + 4)

        @pl.when(g + 3 < nct)
        def _():
            _pref_wait(g + 3)
            pltpu.async_copy(table.at[gidx_a.at[lax.rem(g + 3, 8)]],
                             rows.at[lax.rem(g + 3, 4)],
                             sems.at[lax.rem(g + 3, 4)])

        pltpu.make_async_copy(table.at[gidx_a.at[0]], rows.at[b],
                              sems.at[b]).wait()

        @pl.when(g >= 2)
        def _():
            pltpu.make_async_copy(scbuf.at[sb], acc.at[0].at[sidx_a.at[0]],
                                  ssems.at[sb]).wait()

        def _scale(q, c2):
            svec = vals_a[slot, pl.ds(q * 16, 16)]
            for l in range(16):
                e = q * 16 + l
                s = svec[l]
                scbuf[sb, e, pl.ds(0, 16)] = rows[b, e, pl.ds(0, 16)] * s
                scbuf[sb, e, pl.ds(16, 16)] = rows[b, e, pl.ds(16, 16)] * s
            return c2
        lax.fori_loop(0, C // 16, _scale, 0)

        pltpu.async_copy(scbuf.at[sb], acc.at[i].at[sidx_a.at[slot]],
                         ssems.at[sb], add=True)
        return carry
    lax.fori_loop(0, nct, _chunk, 0)

    for j in range(2):
        pltpu.make_async_copy(scbuf.at[j], acc.at[0].at[sidx_a.at[0]],
                              ssems.at[j]).wait()

    plsc.subcore_barrier()

    for i in range(NS):
        @pl.when(t < N_TILES - 1)
        def _():
            pltpu.sync_copy(acc.at[i].at[pl.ds(start, NR0)],
                            buf.at[pl.ds(0, NR0)])

        @pl.when(t == N_TILES - 1)
        def _():
            pltpu.sync_copy(acc.at[i].at[pl.ds(start, NRL)], buf)

        def _relu(j, carry):
            buf[j, pl.ds(0, 16)] = jnp.maximum(buf[j, pl.ds(0, 16)], zvec)
            buf[j, pl.ds(16, 16)] = jnp.maximum(buf[j, pl.ds(16, 16)], zvec)
            return carry
        lax.fori_loop(0, nrows, _relu, 0)

        col = pl.ds(i * D_SUB, D_SUB)

        @pl.when((d == 0) & (t < N_TILES - 1))
        def _():
            pltpu.sync_copy(buf.at[pl.ds(0, NR0)],
                            outu_hbm.at[pl.ds(start, NR0), col])

        @pl.when((d == 0) & (t == N_TILES - 1))
        def _():
            pltpu.sync_copy(buf, outu_hbm.at[pl.ds(start, NRL), col])

        @pl.when((d == 1) & (t < N_TILES - 1))
        def _():
            pltpu.sync_copy(buf.at[pl.ds(0, NR0)],
                            outv_hbm.at[pl.ds(start, NR0), col])

        @pl.when((d == 1) & (t == N_TILES - 1))
        def _():
            pltpu.sync_copy(buf, outv_hbm.at[pl.ds(start, NRL), col])


def _chunked(a):
    return a.reshape(TOT, C)


def kernel(x_u, x_v, edge_u, edge_v, sup_vals, sup_t_vals, W):
    T = pl.pallas_call(
        _tc_matmul,
        grid=(2,),
        in_specs=[
            pl.BlockSpec((N_U, D_IN), lambda d: (0, 0)),
            pl.BlockSpec((N_V, D_IN), lambda d: (0, 0)),
            pl.BlockSpec((D_IN, D_OUT), lambda d: (0, 0)),
        ],
        out_specs=pl.BlockSpec((1, N_U, D_OUT), lambda d: (d, 0, 0)),
        out_shape=jax.ShapeDtypeStruct((2, N_U, D_OUT), jnp.float32),
    )(x_u, x_v, W)
    T = T.reshape(2, N_U * NS, D_SUB)

    eu = _chunked(edge_u)
    ev = _chunked(edge_v)
    s0 = _chunked(sup_vals)
    s1 = _chunked(sup_t_vals)
    slice_off = (jnp.arange(TOT, dtype=jnp.int32) // CHUNKS)[:, None]
    geu = NS * eu + slice_off
    gev = NS * ev + slice_off

    u_out, v_out = _sc_aggregate(T, eu, ev, geu, gev, s0, s1)
    return (u_out, v_out)


# speedup vs baseline: 1.2489x; 1.2489x over previous
import functools
import jax
import jax.numpy as jnp
import numpy as np
from jax import lax
from jax.experimental import pallas as pl
from jax.experimental.pallas import tpu as pltpu
from jax.experimental.pallas import tpu_sc as plsc

N_U = 10000
N_V = 10000
E = 320000
D_IN = 128
D_OUT = 128
NS = 4
D_SUB = D_OUT // NS
E_BLK = E // NS
C = 128
CHUNKS = E_BLK // C
TOT = NS * CHUNKS
N_TILES = 16
CBIG = 157
CSML = 156
NR0 = 624
NRL = 640


def _tc_matmul(xu_ref, xv_ref, w_ref, out_ref):
    d = pl.program_id(0)

    @pl.when(d == 0)
    def _():
        out_ref[0] = jnp.dot(xv_ref[...], w_ref[...],
                             preferred_element_type=jnp.float32
                             ).astype(jnp.bfloat16)

    @pl.when(d == 1)
    def _():
        out_ref[0] = jnp.dot(xu_ref[...], w_ref[...],
                             preferred_element_type=jnp.float32
                             ).astype(jnp.bfloat16)


_SC_MESH = plsc.VectorSubcoreMesh(core_axis_name="c", subcore_axis_name="s")


@functools.partial(
    pl.kernel,
    out_type=(jax.ShapeDtypeStruct((N_U, D_OUT), jnp.float32),
              jax.ShapeDtypeStruct((N_V, D_OUT), jnp.float32)),
    mesh=_SC_MESH,
    scratch_types=[
        pltpu.VMEM_SHARED((NS, N_U, D_SUB), jnp.float32),
        pltpu.VMEM((3, C), jnp.int32),
        pltpu.VMEM((3, C), jnp.int32),
        pltpu.VMEM((3, C), jnp.float32),
        pltpu.VMEM((2, C, D_SUB), jnp.bfloat16),
        pltpu.VMEM((2, C, D_SUB), jnp.float32),
        pltpu.VMEM((NRL, D_SUB), jnp.float32),
        pltpu.SemaphoreType.DMA((2,)),
        pltpu.SemaphoreType.DMA((2,)),
        pltpu.SemaphoreType.DMA((3, 3)),
    ],
    compiler_params=pltpu.CompilerParams(use_tc_tiling_on_sc=False,
                                         needs_layout_passes=False),
)
def _sc_aggregate(t_hbm, eu_hbm, ev_hbm, geu_hbm, gev_hbm, s0_hbm, s1_hbm,
                  outu_hbm, outv_hbm,
                  acc, gidx_a, sidx_a, vals_a, rows, scbuf, buf,
                  sems, ssems, stsems):
    d = lax.axis_index("c")
    t = lax.axis_index("s")

    zvec = jnp.zeros((16,), jnp.float32)
    start = t * NR0
    nrows = jnp.where(t == N_TILES - 1, NRL, NR0)
    cstart = t * CSML + jnp.minimum(t, 4)
    nct = jnp.where(t < 4, CBIG, CSML)

    def _pref(g):
        slot = lax.rem(g, 3)
        row = cstart + g

        @pl.when(d == 0)
        def _():
            pltpu.async_copy(gev_hbm.at[row], gidx_a.at[slot],
                             stsems.at[slot, 0])
            pltpu.async_copy(eu_hbm.at[row], sidx_a.at[slot],
                             stsems.at[slot, 1])
            pltpu.async_copy(s0_hbm.at[row], vals_a.at[slot],
                             stsems.at[slot, 2])

        @pl.when(d == 1)
        def _():
            pltpu.async_copy(geu_hbm.at[row], gidx_a.at[slot],
                             stsems.at[slot, 0])
            pltpu.async_copy(ev_hbm.at[row], sidx_a.at[slot],
                             stsems.at[slot, 1])
            pltpu.async_copy(s1_hbm.at[row], vals_a.at[slot],
                             stsems.at[slot, 2])

    def _pref_wait(g):
        slot = lax.rem(g, 3)
        pltpu.make_async_copy(eu_hbm.at[0], gidx_a.at[slot],
                              stsems.at[slot, 0]).wait()
        pltpu.make_async_copy(eu_hbm.at[0], sidx_a.at[slot],
                              stsems.at[slot, 1]).wait()
        pltpu.make_async_copy(s0_hbm.at[0], vals_a.at[slot],
                              stsems.at[slot, 2]).wait()

    _pref(0)
    _pref(1)

    def _zero(j, carry):
        buf[j, pl.ds(0, 16)] = zvec
        buf[j, pl.ds(16, 16)] = zvec
        return carry
    lax.fori_loop(0, NRL, _zero, 0)
    for i in range(NS):
        @pl.when(t < N_TILES - 1)
        def _():
            pltpu.sync_copy(buf.at[pl.ds(0, NR0)],
                            acc.at[i].at[pl.ds(start, NR0)])

        @pl.when(t == N_TILES - 1)
        def _():
            pltpu.sync_copy(buf, acc.at[i].at[pl.ds(start, NRL)])
    plsc.subcore_barrier()

    table = t_hbm.at[d]
    _pref_wait(0)
    pltpu.async_copy(table.at[gidx_a.at[0]], rows.at[0], sems.at[0])

    def _chunk(g, carry):
        b = lax.rem(g, 2)
        nb = lax.rem(g + 1, 2)
        slot = lax.rem(g, 3)
        i = lax.div(cstart + g, CHUNKS)

        @pl.when(g + 2 < nct)
        def _():
            _pref(g + 2)

        @pl.when(g + 1 < nct)
        def _():
            _pref_wait(g + 1)
            pltpu.async_copy(table.at[gidx_a.at[lax.rem(g + 1, 3)]],
                             rows.at[nb], sems.at[nb])

        pltpu.make_async_copy(table.at[gidx_a.at[0]], rows.at[b],
                              sems.at[b]).wait()

        @pl.when(g >= 2)
        def _():
            pltpu.make_async_copy(scbuf.at[b], acc.at[0].at[sidx_a.at[0]],
                                  ssems.at[b]).wait()

        def _scale(q, c2):
            svec = vals_a[slot, pl.ds(q * 16, 16)]
            for l in range(16):
                e = q * 16 + l
                s = svec[l]
                lo, hi = plsc.unpack(rows[b, e],
                                     format=plsc.PackFormat.INTERLEAVED,
                                     preferred_element_type=jnp.float32)
                scbuf[b, e, pl.ds(0, 16)] = lo * s
                scbuf[b, e, pl.ds(16, 16)] = hi * s
            return c2
        lax.fori_loop(0, C // 16, _scale, 0)

        pltpu.async_copy(scbuf.at[b], acc.at[i].at[sidx_a.at[slot]],
                         ssems.at[b], add=True)
        return carry
    lax.fori_loop(0, nct, _chunk, 0)

    for j in range(2):
        pltpu.make_async_copy(scbuf.at[j], acc.at[0].at[sidx_a.at[0]],
                              ssems.at[j]).wait()

    plsc.subcore_barrier()

    for i in range(NS):
        @pl.when(t < N_TILES - 1)
        def _():
            pltpu.sync_copy(acc.at[i].at[pl.ds(start, NR0)],
                            buf.at[pl.ds(0, NR0)])

        @pl.when(t == N_TILES - 1)
        def _():
            pltpu.sync_copy(acc.at[i].at[pl.ds(start, NRL)], buf)

        def _relu(j, carry):
            buf[j, pl.ds(0, 16)] = jnp.maximum(buf[j, pl.ds(0, 16)], zvec)
            buf[j, pl.ds(16, 16)] = jnp.maximum(buf[j, pl.ds(16, 16)], zvec)
            return carry
        lax.fori_loop(0, nrows, _relu, 0)

        col = pl.ds(i * D_SUB, D_SUB)

        @pl.when((d == 0) & (t < N_TILES - 1))
        def _():
            pltpu.sync_copy(buf.at[pl.ds(0, NR0)],
                            outu_hbm.at[pl.ds(start, NR0), col])

        @pl.when((d == 0) & (t == N_TILES - 1))
        def _():
            pltpu.sync_copy(buf, outu_hbm.at[pl.ds(start, NRL), col])

        @pl.when((d == 1) & (t < N_TILES - 1))
        def _():
            pltpu.sync_copy(buf.at[pl.ds(0, NR0)],
                            outv_hbm.at[pl.ds(start, NR0), col])

        @pl.when((d == 1) & (t == N_TILES - 1))
        def _():
            pltpu.sync_copy(buf, outv_hbm.at[pl.ds(start, NRL), col])


def _chunked(a):
    return a.reshape(TOT, C)


_COL_PERM = np.concatenate(
    [i * D_SUB + ((np.arange(D_SUB) % 2) * 16 + np.arange(D_SUB) // 2)
     for i in range(NS)])


def kernel(x_u, x_v, edge_u, edge_v, sup_vals, sup_t_vals, W):
    T = pl.pallas_call(
        _tc_matmul,
        grid=(2,),
        in_specs=[
            pl.BlockSpec((N_U, D_IN), lambda d: (0, 0)),
            pl.BlockSpec((N_V, D_IN), lambda d: (0, 0)),
            pl.BlockSpec((D_IN, D_OUT), lambda d: (0, 0)),
        ],
        out_specs=pl.BlockSpec((1, N_U, D_OUT), lambda d: (d, 0, 0)),
        out_shape=jax.ShapeDtypeStruct((2, N_U, D_OUT), jnp.bfloat16),
    )(x_u, x_v, W[:, _COL_PERM])
    T = T.reshape(2, N_U * NS, D_SUB)

    eu = _chunked(edge_u)
    ev = _chunked(edge_v)
    s0 = _chunked(sup_vals)
    s1 = _chunked(sup_t_vals)
    slice_off = (jnp.arange(TOT, dtype=jnp.int32) // CHUNKS)[:, None]
    geu = NS * eu + slice_off
    gev = NS * ev + slice_off

    u_out, v_out = _sc_aggregate(T, eu, ev, geu, gev, s0, s1)
    return (u_out, v_out)


# speedup vs baseline: 1.6235x; 1.3000x over previous
import functools
import jax
import jax.numpy as jnp
from jax import lax
from jax.experimental import pallas as pl
from jax.experimental.pallas import tpu as pltpu
from jax.experimental.pallas import tpu_sc as plsc

N_U = 10000
N_V = 10000
E = 320000
D_IN = 128
D_OUT = 128
NS = 4
D_SUB = D_OUT // NS
E_BLK = E // NS
C = 128
CHUNKS = E_BLK // C
TOT = NS * CHUNKS
N_TILES = 16
CBIG = 157
CSML = 156
NR0 = 624
NRL = 640


def _tc_matmul(xu_ref, xv_ref, w_ref, out_ref):
    d = pl.program_id(0)

    @pl.when(d == 0)
    def _():
        out_ref[0] = jnp.dot(xv_ref[...], w_ref[...],
                             preferred_element_type=jnp.float32)

    @pl.when(d == 1)
    def _():
        out_ref[0] = jnp.dot(xu_ref[...], w_ref[...],
                             preferred_element_type=jnp.float32)


_SC_MESH = plsc.VectorSubcoreMesh(core_axis_name="c", subcore_axis_name="s")


@functools.partial(
    pl.kernel,
    out_type=(jax.ShapeDtypeStruct((N_U, D_OUT), jnp.float32),
              jax.ShapeDtypeStruct((N_V, D_OUT), jnp.float32)),
    mesh=_SC_MESH,
    scratch_types=[
        pltpu.VMEM_SHARED((NS, N_U, D_SUB), jnp.float32),
        pltpu.VMEM((3, C), jnp.int32),
        pltpu.VMEM((3, C), jnp.int32),
        pltpu.VMEM((3, C), jnp.float32),
        pltpu.VMEM((2, C, D_SUB), jnp.float32),
        pltpu.VMEM((NRL, D_SUB), jnp.float32),
        pltpu.SemaphoreType.DMA((2,)),
        pltpu.SemaphoreType.DMA((2,)),
        pltpu.SemaphoreType.DMA((3, 3)),
    ],
    compiler_params=pltpu.CompilerParams(use_tc_tiling_on_sc=False),
)
def _sc_aggregate(t_hbm, eu_hbm, ev_hbm, geu_hbm, gev_hbm, s0_hbm, s1_hbm,
                  outu_hbm, outv_hbm,
                  acc, gidx_a, sidx_a, vals_a, rows, buf,
                  sems, ssems, stsems):
    d = lax.axis_index("c")
    t = lax.axis_index("s")

    zvec = jnp.zeros((16,), jnp.float32)
    start = t * NR0
    nrows = jnp.where(t == N_TILES - 1, NRL, NR0)
    cstart = t * CSML + jnp.minimum(t, 4)
    nct = jnp.where(t < 4, CBIG, CSML)

    def _pref(g):
        slot = lax.rem(g, 3)
        row = cstart + g

        @pl.when(d == 0)
        def _():
            pltpu.async_copy(gev_hbm.at[row], gidx_a.at[slot],
                             stsems.at[slot, 0])
            pltpu.async_copy(eu_hbm.at[row], sidx_a.at[slot],
                             stsems.at[slot, 1])
            pltpu.async_copy(s0_hbm.at[row], vals_a.at[slot],
                             stsems.at[slot, 2])

        @pl.when(d == 1)
        def _():
            pltpu.async_copy(geu_hbm.at[row], gidx_a.at[slot],
                             stsems.at[slot, 0])
            pltpu.async_copy(ev_hbm.at[row], sidx_a.at[slot],
                             stsems.at[slot, 1])
            pltpu.async_copy(s1_hbm.at[row], vals_a.at[slot],
                             stsems.at[slot, 2])

    def _pref_wait(g):
        slot = lax.rem(g, 3)
        pltpu.make_async_copy(eu_hbm.at[0], gidx_a.at[slot],
                              stsems.at[slot, 0]).wait()
        pltpu.make_async_copy(eu_hbm.at[0], sidx_a.at[slot],
                              stsems.at[slot, 1]).wait()
        pltpu.make_async_copy(s0_hbm.at[0], vals_a.at[slot],
                              stsems.at[slot, 2]).wait()

    _pref(0)
    _pref(1)

    def _zero(j, carry):
        buf[j, pl.ds(0, 16)] = zvec
        buf[j, pl.ds(16, 16)] = zvec
        return carry
    lax.fori_loop(0, NRL, _zero, 0)
    for i in range(NS):
        @pl.when(t < N_TILES - 1)
        def _():
            pltpu.sync_copy(buf.at[pl.ds(0, NR0)],
                            acc.at[i].at[pl.ds(start, NR0)])

        @pl.when(t == N_TILES - 1)
        def _():
            pltpu.sync_copy(buf, acc.at[i].at[pl.ds(start, NRL)])
    plsc.subcore_barrier()

    table = t_hbm.at[d]
    _pref_wait(0)
    pltpu.async_copy(table.at[gidx_a.at[0]], rows.at[0], sems.at[0])

    def _chunk(g, carry):
        b = lax.rem(g, 2)
        nb = lax.rem(g + 1, 2)
        slot = lax.rem(g, 3)
        i = lax.div(cstart + g, CHUNKS)

        @pl.when(g >= 1)
        def _():
            pltpu.make_async_copy(rows.at[nb], acc.at[0].at[sidx_a.at[0]],
                                  ssems.at[nb]).wait()

        @pl.when(g + 2 < nct)
        def _():
            _pref(g + 2)

        @pl.when(g + 1 < nct)
        def _():
            _pref_wait(g + 1)
            pltpu.async_copy(table.at[gidx_a.at[lax.rem(g + 1, 3)]],
                             rows.at[nb], sems.at[nb])

        pltpu.make_async_copy(table.at[gidx_a.at[0]], rows.at[b],
                              sems.at[b]).wait()

        def _scale(q, c2):
            svec = vals_a[slot, pl.ds(q * 16, 16)]
            for l in range(16):
                e = q * 16 + l
                s = svec[l]
                rows[b, e, pl.ds(0, 16)] = rows[b, e, pl.ds(0, 16)] * s
                rows[b, e, pl.ds(16, 16)] = rows[b, e, pl.ds(16, 16)] * s
            return c2
        lax.fori_loop(0, C // 16, _scale, 0)

        pltpu.async_copy(rows.at[b], acc.at[i].at[sidx_a.at[slot]],
                         ssems.at[b], add=True)
        return carry
    lax.fori_loop(0, nct, _chunk, 0)

    lastb = lax.rem(nct - 1, 2)
    pltpu.make_async_copy(rows.at[lastb], acc.at[0].at[sidx_a.at[0]],
                          ssems.at[lastb]).wait()

    plsc.subcore_barrier()

    for i in range(NS):
        @pl.when(t < N_TILES - 1)
        def _():
            pltpu.sync_copy(acc.at[i].at[pl.ds(start, NR0)],
                            buf.at[pl.ds(0, NR0)])

        @pl.when(t == N_TILES - 1)
        def _():
            pltpu.sync_copy(acc.at[i].at[pl.ds(start, NRL)], buf)

        def _relu(j, carry):
            buf[j, pl.ds(0, 16)] = jnp.maximum(buf[j, pl.ds(0, 16)], zvec)
            buf[j, pl.ds(16, 16)] = jnp.maximum(buf[j, pl.ds(16, 16)], zvec)
            return carry
        lax.fori_loop(0, nrows, _relu, 0)

        col = pl.ds(i * D_SUB, D_SUB)

        @pl.when((d == 0) & (t < N_TILES - 1))
        def _():
            pltpu.sync_copy(buf.at[pl.ds(0, NR0)],
                            outu_hbm.at[pl.ds(start, NR0), col])

        @pl.when((d == 0) & (t == N_TILES - 1))
        def _():
            pltpu.sync_copy(buf, outu_hbm.at[pl.ds(start, NRL), col])

        @pl.when((d == 1) & (t < N_TILES - 1))
        def _():
            pltpu.sync_copy(buf.at[pl.ds(0, NR0)],
                            outv_hbm.at[pl.ds(start, NR0), col])

        @pl.when((d == 1) & (t == N_TILES - 1))
        def _():
            pltpu.sync_copy(buf, outv_hbm.at[pl.ds(start, NRL), col])


def _chunked(a):
    return a.reshape(TOT, C)


def kernel(x_u, x_v, edge_u, edge_v, sup_vals, sup_t_vals, W):
    T = pl.pallas_call(
        _tc_matmul,
        grid=(2,),
        in_specs=[
            pl.BlockSpec((N_U, D_IN), lambda d: (0, 0)),
            pl.BlockSpec((N_V, D_IN), lambda d: (0, 0)),
            pl.BlockSpec((D_IN, D_OUT), lambda d: (0, 0)),
        ],
        out_specs=pl.BlockSpec((1, N_U, D_OUT), lambda d: (d, 0, 0)),
        out_shape=jax.ShapeDtypeStruct((2, N_U, D_OUT), jnp.float32),
    )(x_u, x_v, W)
    T = T.reshape(2, N_U * NS, D_SUB)

    eu = _chunked(edge_u)
    ev = _chunked(edge_v)
    s0 = _chunked(sup_vals)
    s1 = _chunked(sup_t_vals)
    slice_off = (jnp.arange(TOT, dtype=jnp.int32) // CHUNKS)[:, None]
    geu = NS * eu + slice_off
    gev = NS * ev + slice_off

    u_out, v_out = _sc_aggregate(T, eu, ev, geu, gev, s0, s1)
    return (u_out, v_out)
